# baseline (speedup 1.0000x reference)
import jax
import jax.numpy as jnp
from jax import lax
from jax.experimental import pallas as pl
from jax.experimental.pallas import tpu as pltpu
from jax.experimental.pallas import tpu_sc as plsc

BATCH = 1024
SEQ = 200
HID = 64
LANES = 16
NC = 2
NS = 16
NW = NC * NS
RPW = BATCH // NW
GPR = (SEQ + LANES - 1) // LANES
LAST = SEQ - LANES


def _body(x_hbm, tab_hbm, pos_t_hbm, out_hbm, idx_v, gidx_v, rows_v, obuf_v,
          pos_t_v, gsem):
    wid = lax.axis_index("s") * NC + lax.axis_index("c")
    row0 = wid * RPW

    pltpu.sync_copy(pos_t_hbm, pos_t_v)
    pltpu.sync_copy(x_hbm.at[pl.ds(row0, RPW)], idx_v)

    def chunk_body(c, _):
        for k in range(GPR):
            sl = pl.ds(min(k * LANES, LAST), LANES)
            gidx_v[sl] = lax.shift_right_logical(idx_v[c, sl], 1)

        cps = [
            pltpu.async_copy(
                tab_hbm.at[gidx_v.at[pl.ds(min(k * LANES, LAST), LANES)]],
                rows_v.at[pl.ds(min(k * LANES, LAST), LANES)],
                gsem,
            )
            for k in range(GPR)
        ]
        for cp in cps:
            cp.wait()

        def grp_body(g, _):
            off = jnp.minimum(g * LANES, LAST)
            idx16 = idx_v[c, pl.ds(off, LANES)]
            hoff = (idx16 & 1) * HID
            keep = idx16 != 1
            rowids = off + lax.iota(jnp.int32, LANES)

            def col_body(j, _):
                val = plsc.load_gather(rows_v, [rowids, hoff + j])
                val = jnp.where(keep, val, 0.0) + pos_t_v[j, pl.ds(off, LANES)]
                plsc.store_scatter(
                    obuf_v, [rowids, jnp.full((LANES,), j, jnp.int32)], val
                )
                return 0

            lax.fori_loop(0, HID, col_body, 0)
            return 0

        lax.fori_loop(0, GPR, grp_body, 0)

        pltpu.sync_copy(obuf_v, out_hbm.at[row0 + c])
        return 0

    lax.fori_loop(0, RPW, chunk_body, 0)


@jax.jit
def _run(x, tab2, pos_t):
    mesh = plsc.VectorSubcoreMesh(core_axis_name="c", subcore_axis_name="s")
    f = pl.kernel(
        _body,
        mesh=mesh,
        compiler_params=pltpu.CompilerParams(
            use_tc_tiling_on_sc=True, needs_layout_passes=False
        ),
        out_type=jax.ShapeDtypeStruct((BATCH, SEQ, HID), jnp.float32),
        scratch_types=[
            pltpu.VMEM((RPW, SEQ), jnp.int32),
            pltpu.VMEM((SEQ,), jnp.int32),
            pltpu.VMEM((SEQ, 2 * HID), jnp.float32),
            pltpu.VMEM((SEQ, HID), jnp.float32),
            pltpu.VMEM((HID, SEQ), jnp.float32),
            pltpu.SemaphoreType.DMA,
        ],
    )
    return f(x, tab2, pos_t)


def kernel(x, table, pos_emb):
    tab2 = table.reshape(table.shape[0] // 2, 2 * HID)
    return _run(x.astype(jnp.int32), tab2, pos_emb.T)


# speedup vs baseline: 1.5779x; 1.5779x over previous
import jax
import jax.numpy as jnp
from jax import lax
from jax.experimental import pallas as pl
from jax.experimental.pallas import tpu as pltpu
from jax.experimental.pallas import tpu_sc as plsc

BATCH = 1024
SEQ = 200
HID = 64
LANES = 16
NC = 2
NS = 16
NW = NC * NS
RPW = BATCH // NW
ROWS_PER_CHUNK = 2
CHUNK = ROWS_PER_CHUNK * SEQ
NCHUNK = RPW // ROWS_PER_CHUNK
GPR = (SEQ + LANES - 1) // LANES
LAST = SEQ - LANES


def _body(x_hbm, tab_hbm, pos_hbm, out_hbm, idx_v, rows_v, pos_v, gsem):
    wid = lax.axis_index("s") * NC + lax.axis_index("c")
    row0 = wid * RPW

    pltpu.sync_copy(pos_hbm, pos_v.at[pl.ds(0, SEQ)])
    pltpu.sync_copy(pos_hbm, pos_v.at[pl.ds(SEQ, SEQ)])

    pltpu.sync_copy(x_hbm.at[pl.ds(row0, RPW)], idx_v)

    for c in range(NCHUNK):
        cps = []
        for h in range(ROWS_PER_CHUNK):
            r = c * ROWS_PER_CHUNK + h
            for k in range(GPR):
                off = min(k * LANES, LAST)
                cps.append(
                    pltpu.async_copy(
                        tab_hbm.at[idx_v.at[r, pl.ds(off, LANES)]],
                        rows_v.at[pl.ds(h * SEQ + off, LANES)],
                        gsem,
                    )
                )
        for cp in cps:
            cp.wait()

        for h in range(ROWS_PER_CHUNK):
            r = c * ROWS_PER_CHUNK + h

            def mask_body(g, _, h=h, r=r):
                off = jnp.minimum(g * LANES, LAST)
                v = idx_v[r, pl.ds(off, LANES)]
                m = v == 1
                cnt = jnp.sum(m.astype(jnp.int32))

                @pl.when(cnt > 0)
                def _():
                    ridx = h * SEQ + off + lax.iota(jnp.int32, LANES)
                    zeros = jnp.zeros((LANES,), jnp.float32)

                    def zb(d, _):
                        plsc.store_scatter(
                            rows_v,
                            [ridx, jnp.full((LANES,), d, jnp.int32)],
                            zeros,
                            mask=m,
                        )
                        return 0

                    lax.fori_loop(0, HID, zb, 0)

                return 0

            lax.fori_loop(0, GPR, mask_body, 0)

        def add_body(t, _):
            for d in range(HID // LANES):
                sl = pl.ds(d * LANES, LANES)
                plsc.addupdate(rows_v.at[t, sl], pos_v[t, sl])
            return 0

        lax.fori_loop(0, CHUNK, add_body, 0)

        for h in range(ROWS_PER_CHUNK):
            pltpu.sync_copy(
                rows_v.at[pl.ds(h * SEQ, SEQ)],
                out_hbm.at[row0 + c * ROWS_PER_CHUNK + h],
            )


@jax.jit
def _run(x, table, pos_emb):
    mesh = plsc.VectorSubcoreMesh(core_axis_name="c", subcore_axis_name="s")
    f = pl.kernel(
        _body,
        mesh=mesh,
        compiler_params=pltpu.CompilerParams(
            use_tc_tiling_on_sc=False, needs_layout_passes=False
        ),
        out_type=jax.ShapeDtypeStruct((BATCH, SEQ, HID), jnp.float32),
        scratch_types=[
            pltpu.VMEM((RPW, SEQ), jnp.int32),
            pltpu.VMEM((CHUNK, HID), jnp.float32),
            pltpu.VMEM((CHUNK, HID), jnp.float32),
            pltpu.SemaphoreType.DMA,
        ],
    )
    return f(x, table, pos_emb)


def kernel(x, table, pos_emb):
    return _run(x.astype(jnp.int32), table, pos_emb)
